# Initial kernel scaffold; baseline (speedup 1.0000x reference)
import functools

import jax
import jax.numpy as jnp
import numpy as np
from jax.experimental import pallas as pl
from jax.experimental.pallas import tpu as pltpu
from jax.experimental.pallas import tpu_sc as plsc

B, H, V = 128, 4, 100000
N = B * H
M = 128
NC, NS = 2, 16
NW = NC * NS
RPW = N // NW
CHUNK = 20000
NCH = V // CHUNK
GROUP = 80
PRIME = 240
CAP = 512
VLEN = 608
_SAMPLING_EPS = 1e-5
_TINY = float(np.finfo(np.float32).tiny)

_mesh = plsc.VectorSubcoreMesh(core_axis_name="c", subcore_axis_name="s")


def _key_of(vf):
    b = jax.lax.bitcast_convert_type(vf, jnp.uint32)
    m = jnp.where(b >= jnp.uint32(0x80000000), jnp.uint32(0xFFFFFFFF),
                  jnp.uint32(0x80000000))
    return b ^ m


def _unkey(k):
    b = jnp.where(k >= jnp.uint32(0x80000000), k ^ jnp.uint32(0x80000000),
                  k ^ jnp.uint32(0xFFFFFFFF))
    return jax.lax.bitcast_convert_type(b, jnp.float32)


@functools.partial(
    pl.kernel,
    mesh=_mesh,
    out_type=[jax.ShapeDtypeStruct((N * M,), jnp.float32),
              jax.ShapeDtypeStruct((N * M,), jnp.int32)],
    scratch_types=[pltpu.VMEM((CHUNK,), jnp.float32),
                   pltpu.VMEM((VLEN,), jnp.float32),
                   pltpu.VMEM((VLEN,), jnp.int32),
                   pltpu.VMEM((VLEN,), jnp.uint32),
                   pltpu.VMEM((VLEN,), jnp.float32),
                   pltpu.VMEM((VLEN,), jnp.int32)],
)
def _phase1(l_hbm, outv_hbm, outi_hbm, stage, vbuf, ibuf, kbuf, cvb, cib):
    wid = jax.lax.axis_index("s") * NC + jax.lax.axis_index("c")
    iota = jax.lax.iota(jnp.int32, 16)
    ninf = jnp.full((16,), -jnp.inf, jnp.float32)

    def count_ge(tkey, n):
        nv = (n + 15) // 16
        def cb(j, acc):
            k = kbuf[pl.ds(j * 16, 16)]
            live = (iota + j * 16) < n
            return acc + jnp.where(live & (k >= tkey), 1, 0)
        acc = jax.lax.fori_loop(0, nv, cb, jnp.zeros((16,), jnp.int32))
        return jnp.sum(acc)

    def prune(n, tau, lo_t, hi_t, into_out):
        def kb(j, _):
            kbuf[pl.ds(j * 16, 16)] = _key_of(vbuf[pl.ds(j * 16, 16)])
            return 0
        jax.lax.fori_loop(0, (n + 15) // 16, kb, 0)

        def cond(st):
            lo, hi, cnt, it = st
            return (cnt > hi_t) & (it < 34)

        def body(st):
            lo, hi, cnt, it = st
            mid = lo + ((hi - lo) >> jnp.uint32(1))
            cm = count_ge(mid, n)
            ge = cm >= lo_t
            lo = jnp.where(ge, mid, lo)
            hi = jnp.where(ge, hi, mid)
            cnt = jnp.where(ge, cm, cnt)
            return lo, hi, cnt, it + 1

        lo0 = _key_of(tau)
        hi0 = jnp.full((16,), 0xFFFFFFFF, jnp.uint32)
        lo, _, _, _ = jax.lax.while_loop(cond, body, (lo0, hi0, n, 0))
        tau2 = _unkey(lo)

        def comp(j, nn):
            v = vbuf[pl.ds(j * 16, 16)]
            ii = ibuf[pl.ds(j * 16, 16)]
            m = (((iota + j * 16) < n) & (v >= tau2))
            if into_out:
                plsc.store_compressed(cvb.at[pl.ds(nn, 16)], v, m)
                plsc.store_compressed(cib.at[pl.ds(nn, 16)], ii, m)
            else:
                plsc.store_compressed(vbuf.at[pl.ds(nn, 16)], v, m)
                plsc.store_compressed(ibuf.at[pl.ds(nn, 16)], ii, m)
            return nn + jnp.sum(m.astype(jnp.int32))

        n2 = jax.lax.fori_loop(0, (n + 15) // 16, comp, 0)
        return n2, tau2

    def row_body(r, _):
        row = wid * RPW + r
        rowbase = row * V

        pltpu.sync_copy(l_hbm.at[pl.ds(rowbase, CHUNK)], stage)

        def pb(j, _):
            vbuf[pl.ds(j * 16, 16)] = stage[pl.ds(j * 16, 16)]
            ibuf[pl.ds(j * 16, 16)] = iota + j * 16
            return 0
        jax.lax.fori_loop(0, PRIME // 16, pb, 0)
        n, tau = prune(PRIME, ninf, 128, 320, False)

        def mk_group(cbase):
            def group(g, carry):
                n, tau = carry
                off = g * GROUP
                vs = [stage[pl.ds(off + 16 * i, 16)] for i in range(5)]
                mx = jnp.maximum(jnp.maximum(jnp.maximum(vs[0], vs[1]),
                                             jnp.maximum(vs[2], vs[3])), vs[4])

                def fire(n, tau):
                    for i in range(5):
                        m = vs[i] >= tau
                        def app(nn, i=i, m=m):
                            idxv = iota + (cbase + 16 * i) + off
                            plsc.store_compressed(vbuf.at[pl.ds(nn, 16)], vs[i], m)
                            plsc.store_compressed(ibuf.at[pl.ds(nn, 16)], idxv, m)
                            return nn + jnp.sum(m.astype(jnp.int32))
                        n = jax.lax.cond(jnp.any(m), app, lambda nn: nn, n)
                    return jax.lax.cond(n >= CAP,
                                        lambda a, b: prune(a, b, 128, 320, False),
                                        lambda a, b: (a, b), n, tau)

                return jax.lax.cond(jnp.any(mx >= tau), fire,
                                    lambda a, b: (a, b), n, tau)
            return group

        for c in range(NCH):
            if c > 0:
                pltpu.sync_copy(l_hbm.at[pl.ds(rowbase + c * CHUNK, CHUNK)], stage)
            start = PRIME // GROUP if c == 0 else 0
            n, tau = jax.lax.fori_loop(start, CHUNK // GROUP,
                                       mk_group(c * CHUNK), (n, tau))

        def pad(j, _):
            cvb[pl.ds(j * 16, 16)] = ninf
            cib[pl.ds(j * 16, 16)] = jnp.full((16,), -1, jnp.int32)
            return 0
        jax.lax.fori_loop(0, (M + 16) // 16, pad, 0)
        prune(n, tau, 99, 128, True)
        pltpu.sync_copy(cvb.at[pl.ds(0, M)], outv_hbm.at[pl.ds(row * M, M)])
        pltpu.sync_copy(cib.at[pl.ds(0, M)], outi_hbm.at[pl.ds(row * M, M)])
        return 0

    jax.lax.fori_loop(0, RPW, row_body, 0)


def _tf_rotl(x, d):
    return (x << jnp.uint32(d)) | (x >> jnp.uint32(32 - d))


def _threefry2x32(c1):
    k0 = jnp.uint32(0)
    k1 = jnp.uint32(42)
    ks = [k0, k1, k0 ^ k1 ^ jnp.uint32(0x1BD11BDA)]
    rot = [[13, 15, 26, 6], [17, 29, 16, 24]]
    x0 = jnp.zeros_like(c1) + ks[0]
    x1 = c1 + ks[1]
    for g in range(5):
        for r in rot[g % 2]:
            x0 = x0 + x1
            x1 = _tf_rotl(x1, r)
            x1 = x1 ^ x0
        x0 = x0 + ks[(g + 1) % 3]
        x1 = x1 + ks[(g + 2) % 3] + jnp.uint32(g + 1)
    return x0 ^ x1


_P2R = 32


def _phase2_body(cv_ref, ci_ref, t_ref, p_ref, k_ref, po_ref, ids_ref):
    x0 = cv_ref[...]
    idx = ci_ref[...]
    valid = idx >= 0
    temp = t_ref[...]
    temp = jnp.where(temp < _SAMPLING_EPS, 1.0, temp)
    x = x0 / temp
    x = jnp.where(valid, x, -jnp.inf)
    xl = x[:, None, :]
    xj = x[:, :, None]
    n_gt = jnp.sum((xl > xj).astype(jnp.float32), axis=2)
    kk = jnp.clip(k_ref[...], 1, V).astype(jnp.float32)
    keep1 = (n_gt < kk) & valid
    x1 = jnp.where(keep1, x, -jnp.inf)
    mx = jnp.max(x1, axis=1, keepdims=True)
    e = jnp.exp(x1 - mx)
    s = e / jnp.sum(e, axis=1, keepdims=True)
    x1l = x1[:, None, :]
    x1j = x1[:, :, None]
    before = (x1l < x1j) | ((x1l == x1j) & (idx[:, None, :] <= idx[:, :, None]))
    c = jnp.sum(jnp.where(before, s[:, None, :], 0.0), axis=2)
    pmask = c <= (1.0 - p_ref[...])
    ismax = x1 == mx
    lastidx = jnp.max(jnp.where(ismax, idx, -1), axis=1, keepdims=True)
    force = ismax & (idx == lastidx)
    x2 = jnp.where(pmask & ~force, -jnp.inf, x1)
    mx2 = jnp.max(x2, axis=1, keepdims=True)
    e2 = jnp.exp(x2 - mx2)
    pfin = e2 / jnp.sum(e2, axis=1, keepdims=True)
    pfin = jnp.where(valid, pfin, 0.0)
    po_ref[...] = pfin

    row = pl.program_id(0) * _P2R + jax.lax.broadcasted_iota(jnp.int32, (_P2R, M), 0)
    flat = row * V + jnp.where(valid, idx, 0)
    bits = _threefry2x32(flat.astype(jnp.uint32))
    f = jax.lax.bitcast_convert_type((bits >> jnp.uint32(9))
                                     | jnp.uint32(0x3F800000), jnp.float32) - 1.0
    u = jnp.maximum(_TINY, f * (1.0 - _TINY) + _TINY)
    g = -jnp.log(-jnp.log(u))
    w = g + jnp.log(pfin + 1e-30)
    w = jnp.where(valid, w, -jnp.inf)
    wmax = jnp.max(w, axis=1, keepdims=True)
    sel = (w == wmax) & valid
    ids_ref[...] = jnp.min(jnp.where(sel, idx, V), axis=1, keepdims=True)


def _phase2(cv, ci, temps, ps, ks):
    grid = N // _P2R
    return pl.pallas_call(
        _phase2_body,
        grid=(grid,),
        in_specs=[pl.BlockSpec((_P2R, M), lambda i: (i, 0)),
                  pl.BlockSpec((_P2R, M), lambda i: (i, 0)),
                  pl.BlockSpec((_P2R, 1), lambda i: (i, 0)),
                  pl.BlockSpec((_P2R, 1), lambda i: (i, 0)),
                  pl.BlockSpec((_P2R, 1), lambda i: (i, 0))],
        out_specs=[pl.BlockSpec((_P2R, M), lambda i: (i, 0)),
                   pl.BlockSpec((_P2R, 1), lambda i: (i, 0))],
        out_shape=[jax.ShapeDtypeStruct((N, M), jnp.float32),
                   jax.ShapeDtypeStruct((N, 1), jnp.int32)],
    )(cv, ci, temps, ps, ks)


@functools.partial(
    pl.kernel,
    mesh=_mesh,
    out_type=jax.ShapeDtypeStruct((N * V,), jnp.float32),
    scratch_types=[pltpu.VMEM((V,), jnp.float32),
                   pltpu.VMEM((M,), jnp.float32),
                   pltpu.VMEM((M,), jnp.int32)],
)
def _phase3(p_hbm, ci_hbm, probs_hbm, zbuf, pst, ist):
    wid = jax.lax.axis_index("s") * NC + jax.lax.axis_index("c")
    zero = jnp.zeros((16,), jnp.float32)

    def zb(j, _):
        zbuf[pl.ds(j * 16, 16)] = zero
        return 0
    jax.lax.fori_loop(0, V // 16, zb, 0)

    def row_body(r, _):
        row = wid * RPW + r
        pltpu.sync_copy(p_hbm.at[pl.ds(row * M, M)], pst)
        pltpu.sync_copy(ci_hbm.at[pl.ds(row * M, M)], ist)
        for j in range(M // 16):
            iv = ist[pl.ds(j * 16, 16)]
            pv = pst[pl.ds(j * 16, 16)]
            plsc.store_scatter(zbuf, [iv], pv, iv >= 0)
        pltpu.sync_copy(zbuf, probs_hbm.at[pl.ds(row * V, V)])
        for j in range(M // 16):
            iv = ist[pl.ds(j * 16, 16)]
            plsc.store_scatter(zbuf, [iv], zero, iv >= 0)
        return 0

    jax.lax.fori_loop(0, RPW, row_body, 0)


def kernel(logits, temperatures, top_ps, top_ks):
    lflat = logits.astype(jnp.float32).reshape(N * V)
    cv, ci = _phase1(lflat)
    p, ids = _phase2(cv.reshape(N, M), ci.reshape(N, M),
                     temperatures.reshape(N, 1).astype(jnp.float32),
                     top_ps.reshape(N, 1).astype(jnp.float32),
                     top_ks.reshape(N, 1).astype(jnp.int32))
    probs = _phase3(p.reshape(N * M), ci)
    return ids.reshape(B, H), probs.reshape(N, V)


# speedup vs baseline: 62.9908x; 62.9908x over previous
import functools

import jax
import jax.numpy as jnp
import numpy as np
from jax.experimental import pallas as pl
from jax.experimental.pallas import tpu as pltpu
from jax.experimental.pallas import tpu_sc as plsc

B, H, V = 128, 4, 100000
N = B * H
M = 128
NC, NS = 2, 16
NW = NC * NS
RPW = N // NW
CHUNK = 20000
NCH = V // CHUNK
GROUP = 80
PRIME = 240
CAP = 512
VLEN = 640
_SAMPLING_EPS = 1e-5
_TINY = float(np.finfo(np.float32).tiny)

_sc_cache = {}


def _sc_mesh():
    return plsc.VectorSubcoreMesh(core_axis_name="c", subcore_axis_name="s")


_IOTA16 = None


_GDN = jax.lax.GatherDimensionNumbers(offset_dims=(), collapsed_slice_dims=(0,),
                                      start_index_map=(0,))


def _perm16(v, idx):
    return jax.lax.gather(v, idx[:, None], _GDN, (1,),
                          mode=jax.lax.GatherScatterMode.PROMISE_IN_BOUNDS)


def _hsplat(v):
    iota = jax.lax.iota(jnp.int32, 16)
    for d in (1, 2, 4, 8):
        v = v + _perm16(v, iota ^ d)
    return v


def _key_of(vf):
    b = jax.lax.bitcast_convert_type(vf, jnp.uint32)
    m = jnp.where(b >= jnp.uint32(0x80000000), jnp.uint32(0xFFFFFFFF),
                  jnp.uint32(0x80000000))
    return b ^ m


def _unkey(k):
    b = jnp.where(k >= jnp.uint32(0x80000000), k ^ jnp.uint32(0x80000000),
                  k ^ jnp.uint32(0xFFFFFFFF))
    return jax.lax.bitcast_convert_type(b, jnp.float32)


def _phase1(lflat):
    if "p1" not in _sc_cache:
        _sc_cache["p1"] = functools.partial(
            pl.kernel,
            mesh=_sc_mesh(),
            compiler_params=pltpu.CompilerParams(needs_layout_passes=False),
            out_type=[jax.ShapeDtypeStruct((N * M,), jnp.float32),
                      jax.ShapeDtypeStruct((N * M,), jnp.int32)],
            scratch_types=[pltpu.VMEM((20096,), jnp.float32),
                           pltpu.VMEM((VLEN,), jnp.float32),
                           pltpu.VMEM((VLEN,), jnp.int32),
                           pltpu.VMEM((VLEN,), jnp.uint32),
                           pltpu.VMEM((VLEN,), jnp.float32),
                           pltpu.VMEM((VLEN,), jnp.int32)],
        )(_phase1_body)
    return _sc_cache["p1"](lflat)


def _phase1_body(l_hbm, outv_hbm, outi_hbm, stage, vbuf, ibuf, kbuf, cvb, cib):
    wid = jax.lax.axis_index("s") * NC + jax.lax.axis_index("c")
    iota = jax.lax.iota(jnp.int32, 16)
    ninf = jnp.full((16,), -jnp.inf, jnp.float32)

    def count_ge(tkey, n):
        nv = (n + 15) // 16
        def cb(j, acc):
            k = kbuf[pl.ds(j * 16, 16)]
            live = (iota + j * 16) < n
            return acc + jnp.where(live & (k >= tkey), 1, 0)
        acc = jax.lax.fori_loop(0, nv, cb, jnp.zeros((16,), jnp.int32))
        return _hsplat(acc)

    def prune(n, tau, lo_t, hi_t, into_out):
        def kb(j, _):
            kbuf[pl.ds(j * 16, 16)] = _key_of(vbuf[pl.ds(j * 16, 16)])
            return 0
        jax.lax.fori_loop(0, (n + 15) // 16, kb, 0)

        def bis(_, st):
            lo, hi = st
            mid = lo + ((hi - lo) >> jnp.uint32(1))
            cm = count_ge(mid, n)
            ge = cm >= lo_t
            lo = jnp.where(ge, mid, lo)
            hi = jnp.where(ge, hi, mid)
            return lo, hi

        lo0 = _key_of(tau)
        hi0 = jnp.full((16,), 0xFFFFFFFF, jnp.uint32)
        lo, _ = jax.lax.fori_loop(0, 22, bis, (lo0, hi0))
        tau2 = _unkey(lo)

        def comp(j, nn):
            v = vbuf[pl.ds(j * 16, 16)]
            ii = ibuf[pl.ds(j * 16, 16)]
            m = (((iota + j * 16) < n) & (v >= tau2))
            if into_out:
                plsc.store_compressed(cvb.at[pl.ds(nn, 16)], v, mask=m)
                plsc.store_compressed(cib.at[pl.ds(nn, 16)], ii, mask=m)
            else:
                plsc.store_compressed(vbuf.at[pl.ds(nn, 16)], v, mask=m)
                plsc.store_compressed(ibuf.at[pl.ds(nn, 16)], ii, mask=m)
            return nn + _hsplat(m.astype(jnp.int32))[0]

        n2 = jax.lax.fori_loop(0, (n + 15) // 16, comp, 0)
        return n2, tau2

    def row_body(r, _):
        row = wid * RPW + r
        rowbase = row * V

        pltpu.sync_copy(l_hbm.at[pl.ds(rowbase, CHUNK)], stage.at[pl.ds(0, CHUNK)])

        def pb(j, _):
            vbuf[pl.ds(j * 16, 16)] = stage[pl.ds(j * 16, 16)]
            ibuf[pl.ds(j * 16, 16)] = iota + j * 16
            return 0
        jax.lax.fori_loop(0, PRIME // 16, pb, 0)
        n, tau = prune(PRIME, ninf, 128, 320, False)

        def mk_group(cbase):
            def group(g, carry):
                n, tau = carry
                off = g * GROUP
                vs = [stage[pl.ds(off + 16 * i, 16)] for i in range(5)]
                mx = jnp.maximum(jnp.maximum(jnp.maximum(vs[0], vs[1]),
                                             jnp.maximum(vs[2], vs[3])), vs[4])

                def fire(n, tau):
                    for i in range(5):
                        m = vs[i] >= tau
                        def app(nn, i=i, m=m):
                            idxv = iota + (cbase + 16 * i) + off
                            plsc.store_compressed(vbuf.at[pl.ds(nn, 16)], vs[i], mask=m)
                            plsc.store_compressed(ibuf.at[pl.ds(nn, 16)], idxv, mask=m)
                            return nn + _hsplat(m.astype(jnp.int32))[0]
                        n = jax.lax.cond(jnp.any(m), app, lambda nn: nn, n)
                    return jax.lax.cond(n >= CAP,
                                        lambda a, b: prune(a, b, 128, 320, False),
                                        lambda a, b: (a, b), n, tau)

                return jax.lax.cond(jnp.any(mx >= tau), fire,
                                    lambda a, b: (a, b), n, tau)
            return group

        for c in range(NCH):
            if c > 0:
                pltpu.sync_copy(l_hbm.at[pl.ds(rowbase + c * CHUNK, CHUNK)], stage.at[pl.ds(0, CHUNK)])
            start = PRIME // GROUP if c == 0 else 0
            n, tau = jax.lax.fori_loop(start, CHUNK // GROUP,
                                       mk_group(c * CHUNK), (n, tau))

        def pad(j, _):
            cvb[pl.ds(j * 16, 16)] = ninf
            cib[pl.ds(j * 16, 16)] = jnp.full((16,), -1, jnp.int32)
            return 0
        jax.lax.fori_loop(0, (M + 16) // 16, pad, 0)
        prune(n, tau, 99, 128, True)
        pltpu.sync_copy(cvb.at[pl.ds(0, M)], outv_hbm.at[pl.ds(row * M, M)])
        pltpu.sync_copy(cib.at[pl.ds(0, M)], outi_hbm.at[pl.ds(row * M, M)])
        return 0

    jax.lax.fori_loop(0, RPW, row_body, 0)


def _tf_rotl(x, d):
    return (x << jnp.uint32(d)) | (x >> jnp.uint32(32 - d))


def _threefry2x32(c1):
    k0 = jnp.uint32(0)
    k1 = jnp.uint32(42)
    ks = [k0, k1, k0 ^ k1 ^ jnp.uint32(0x1BD11BDA)]
    rot = [[13, 15, 26, 6], [17, 29, 16, 24]]
    x0 = jnp.zeros_like(c1) + ks[0]
    x1 = c1 + ks[1]
    for g in range(5):
        for r in rot[g % 2]:
            x0 = x0 + x1
            x1 = _tf_rotl(x1, r)
            x1 = x1 ^ x0
        x0 = x0 + ks[(g + 1) % 3]
        x1 = x1 + ks[(g + 2) % 3] + jnp.uint32(g + 1)
    return x0 ^ x1


_P2R = 128


def _phase2_body(cv_ref, ci_ref, t_ref, p_ref, k_ref, po_ref, ids_ref):
    x0 = cv_ref[...]
    idx = ci_ref[...]
    valid = idx >= 0
    temp = t_ref[...]
    temp = jnp.where(temp < _SAMPLING_EPS, 1.0, temp)
    x = x0 / temp
    x = jnp.where(valid, x, -jnp.inf)
    kk = jnp.clip(k_ref[...], 1, V).astype(jnp.float32)

    n_gt = jnp.zeros_like(x)
    for l in range(M):
        n_gt = n_gt + (x[:, l:l + 1] > x).astype(jnp.float32)
    keep1 = (n_gt < kk) & valid
    x1 = jnp.where(keep1, x, -jnp.inf)
    mx = jnp.max(x1, axis=1, keepdims=True)
    e = jnp.exp(x1 - mx)
    s = e / jnp.sum(e, axis=1, keepdims=True)

    c = jnp.zeros_like(x)
    for l in range(M):
        xl = x1[:, l:l + 1]
        il = idx[:, l:l + 1]
        sl = s[:, l:l + 1]
        before = (xl < x1) | ((xl == x1) & (il <= idx))
        c = c + jnp.where(before, sl, 0.0)
    pmask = c <= (1.0 - p_ref[...])
    ismax = x1 == mx
    lastidx = jnp.max(jnp.where(ismax, idx, -1), axis=1, keepdims=True)
    force = ismax & (idx == lastidx)
    x2 = jnp.where(pmask & ~force, -jnp.inf, x1)
    mx2 = jnp.max(x2, axis=1, keepdims=True)
    e2 = jnp.exp(x2 - mx2)
    pfin = e2 / jnp.sum(e2, axis=1, keepdims=True)
    pfin = jnp.where(valid, pfin, 0.0)
    po_ref[...] = pfin

    row = pl.program_id(0) * _P2R + jax.lax.broadcasted_iota(jnp.int32, (_P2R, M), 0)
    flat = row * V + jnp.where(valid, idx, 0)
    bits = _threefry2x32(flat.astype(jnp.uint32))
    f = jax.lax.bitcast_convert_type((bits >> jnp.uint32(9))
                                     | jnp.uint32(0x3F800000), jnp.float32) - 1.0
    u = jnp.maximum(_TINY, f * (1.0 - _TINY) + _TINY)
    g = -jnp.log(-jnp.log(u))
    w = g + jnp.log(pfin + 1e-30)
    w = jnp.where(valid, w, -jnp.inf)
    wmax = jnp.max(w, axis=1, keepdims=True)
    sel = (w == wmax) & valid
    ids_ref[...] = jnp.min(jnp.where(sel, idx, V), axis=1, keepdims=True)


def _phase2(cv, ci, temps, ps, ks):
    grid = N // _P2R
    return pl.pallas_call(
        _phase2_body,
        grid=(grid,),
        in_specs=[pl.BlockSpec((_P2R, M), lambda i: (i, 0)),
                  pl.BlockSpec((_P2R, M), lambda i: (i, 0)),
                  pl.BlockSpec((_P2R, 1), lambda i: (i, 0)),
                  pl.BlockSpec((_P2R, 1), lambda i: (i, 0)),
                  pl.BlockSpec((_P2R, 1), lambda i: (i, 0))],
        out_specs=[pl.BlockSpec((_P2R, M), lambda i: (i, 0)),
                   pl.BlockSpec((_P2R, 1), lambda i: (i, 0))],
        out_shape=[jax.ShapeDtypeStruct((N, M), jnp.float32),
                   jax.ShapeDtypeStruct((N, 1), jnp.int32)],
    )(cv, ci, temps, ps, ks)


def _phase3(p_flat, ci):
    if "p3" not in _sc_cache:
        _sc_cache["p3"] = functools.partial(
            pl.kernel,
            mesh=_sc_mesh(),
            compiler_params=pltpu.CompilerParams(needs_layout_passes=False),
            out_type=jax.ShapeDtypeStruct((N * V,), jnp.float32),
            scratch_types=[pltpu.VMEM((100096,), jnp.float32),
                           pltpu.VMEM((M,), jnp.float32),
                           pltpu.VMEM((M,), jnp.int32)],
        )(_phase3_body)
    return _sc_cache["p3"](p_flat, ci)


def _phase3_body(p_hbm, ci_hbm, probs_hbm, zbuf, pst, ist):
    wid = jax.lax.axis_index("s") * NC + jax.lax.axis_index("c")
    zero = jnp.zeros((16,), jnp.float32)

    def zb(j, _):
        zbuf[pl.ds(j * 16, 16)] = zero
        return 0
    jax.lax.fori_loop(0, 100096 // 16, zb, 0)

    def row_body(r, _):
        row = wid * RPW + r
        pltpu.sync_copy(p_hbm.at[pl.ds(row * M, M)], pst)
        pltpu.sync_copy(ci_hbm.at[pl.ds(row * M, M)], ist)
        for j in range(M // 16):
            iv = ist[pl.ds(j * 16, 16)]
            pv = pst[pl.ds(j * 16, 16)]
            plsc.store_scatter(zbuf, [iv], pv, mask=iv >= 0)
        pltpu.sync_copy(zbuf.at[pl.ds(0, V)], probs_hbm.at[pl.ds(row * V, V)])
        for j in range(M // 16):
            iv = ist[pl.ds(j * 16, 16)]
            plsc.store_scatter(zbuf, [iv], zero, mask=iv >= 0)
        return 0

    jax.lax.fori_loop(0, RPW, row_body, 0)


def kernel(logits, temperatures, top_ps, top_ks):
    lflat = logits.astype(jnp.float32).reshape(N * V)
    cv, ci = _phase1(lflat)
    p, ids = _phase2(cv.reshape(N, M), ci.reshape(N, M),
                     temperatures.reshape(N, 1).astype(jnp.float32),
                     top_ps.reshape(N, 1).astype(jnp.float32),
                     top_ks.reshape(N, 1).astype(jnp.int32))
    probs = _phase3(p.reshape(N * M), ci)
    return ids.reshape(B, H), probs.reshape(N, V)


# speedup vs baseline: 128.3538x; 2.0377x over previous
import functools

import jax
import jax.numpy as jnp
import numpy as np
from jax.experimental import pallas as pl
from jax.experimental.pallas import tpu as pltpu
from jax.experimental.pallas import tpu_sc as plsc

B, H, V = 128, 4, 100000
N = B * H
M = 128
NC, NS = 2, 16
NW = NC * NS
RPW = N // NW
CHUNK = 20000
NCH = V // CHUNK
GROUP = 80
PRIME = 640
CAP = 512
VLEN = 640
_SAMPLING_EPS = 1e-5
_TINY = float(np.finfo(np.float32).tiny)

_sc_cache = {}


def _sc_mesh():
    return plsc.VectorSubcoreMesh(core_axis_name="c", subcore_axis_name="s")


_IOTA16 = None


_GDN = jax.lax.GatherDimensionNumbers(offset_dims=(), collapsed_slice_dims=(0,),
                                      start_index_map=(0,))


def _perm16(v, idx):
    return jax.lax.gather(v, idx[:, None], _GDN, (1,),
                          mode=jax.lax.GatherScatterMode.PROMISE_IN_BOUNDS)


def _popcnt(m):
    return plsc.all_reduce_population_count(m)[0]


def _hsplat(v):
    iota = jax.lax.iota(jnp.int32, 16)
    for d in (1, 2, 4, 8):
        v = v + _perm16(v, iota ^ d)
    return v


def _key_of(vf):
    b = jax.lax.bitcast_convert_type(vf, jnp.uint32)
    m = jnp.where(b >= jnp.uint32(0x80000000), jnp.uint32(0xFFFFFFFF),
                  jnp.uint32(0x80000000))
    return b ^ m


def _unkey(k):
    b = jnp.where(k >= jnp.uint32(0x80000000), k ^ jnp.uint32(0x80000000),
                  k ^ jnp.uint32(0xFFFFFFFF))
    return jax.lax.bitcast_convert_type(b, jnp.float32)


def _phase1(lflat):
    if "p1" not in _sc_cache:
        _sc_cache["p1"] = functools.partial(
            pl.kernel,
            mesh=_sc_mesh(),
            compiler_params=pltpu.CompilerParams(needs_layout_passes=False),
            out_type=[jax.ShapeDtypeStruct((N * M,), jnp.float32),
                      jax.ShapeDtypeStruct((N * M,), jnp.int32)],
            scratch_types=[pltpu.VMEM((20096,), jnp.float32),
                           pltpu.VMEM((VLEN,), jnp.float32),
                           pltpu.VMEM((VLEN,), jnp.int32),
                           pltpu.VMEM((VLEN,), jnp.uint32),
                           pltpu.VMEM((VLEN,), jnp.float32),
                           pltpu.VMEM((VLEN,), jnp.int32)],
        )(_phase1_body)
    return _sc_cache["p1"](lflat)


def _phase1_body(l_hbm, outv_hbm, outi_hbm, stage, vbuf, ibuf, kbuf, cvb, cib):
    wid = jax.lax.axis_index("s") * NC + jax.lax.axis_index("c")
    iota = jax.lax.iota(jnp.int32, 16)
    ninf = jnp.full((16,), -jnp.inf, jnp.float32)

    def count_ge(tkey, n):
        nv = (n + 15) // 16
        def cb(j, acc):
            k = kbuf[pl.ds(j * 16, 16)]
            live = (iota + j * 16) < n
            return acc + jnp.where(live & (k >= tkey), 1, 0)
        acc = jax.lax.fori_loop(0, nv, cb, jnp.zeros((16,), jnp.int32))
        return _hsplat(acc)

    def prune(n, tau, lo_t, hi_t, into_out):
        def kb(j, _):
            kbuf[pl.ds(j * 16, 16)] = _key_of(vbuf[pl.ds(j * 16, 16)])
            return 0
        jax.lax.fori_loop(0, (n + 15) // 16, kb, 0)

        def bis(_, st):
            lo, hi = st
            mid = lo + ((hi - lo) >> jnp.uint32(1))
            cm = count_ge(mid, n)
            ge = cm >= lo_t
            lo = jnp.where(ge, mid, lo)
            hi = jnp.where(ge, hi, mid)
            return lo, hi

        lo0 = _key_of(tau)
        hi0 = jnp.full((16,), 0xFFFFFFFF, jnp.uint32)
        lo, _ = jax.lax.fori_loop(0, 22, bis, (lo0, hi0))
        tau2 = _unkey(lo)

        def comp(j, nn):
            v = vbuf[pl.ds(j * 16, 16)]
            ii = ibuf[pl.ds(j * 16, 16)]
            m = (((iota + j * 16) < n) & (v >= tau2))
            if into_out:
                plsc.store_compressed(cvb.at[pl.ds(nn, 16)], v, mask=m)
                plsc.store_compressed(cib.at[pl.ds(nn, 16)], ii, mask=m)
            else:
                plsc.store_compressed(vbuf.at[pl.ds(nn, 16)], v, mask=m)
                plsc.store_compressed(ibuf.at[pl.ds(nn, 16)], ii, mask=m)
            return nn + _popcnt(m)

        n2 = jax.lax.fori_loop(0, (n + 15) // 16, comp, 0)
        return n2, tau2

    def row_body(r, _):
        row = wid * RPW + r
        rowbase = row * V

        pltpu.sync_copy(l_hbm.at[pl.ds(rowbase, CHUNK)], stage.at[pl.ds(0, CHUNK)])

        def pb(j, _):
            vbuf[pl.ds(j * 16, 16)] = stage[pl.ds(j * 16, 16)]
            ibuf[pl.ds(j * 16, 16)] = iota + j * 16
            return 0
        jax.lax.fori_loop(0, PRIME // 16, pb, 0)
        n, tau = prune(PRIME, ninf, 128, 320, False)

        def mk_group(cbase):
            def group(g, carry):
                n, tau = carry
                off = g * GROUP
                vs = [stage[pl.ds(off + 16 * i, 16)] for i in range(5)]
                mx = jnp.maximum(jnp.maximum(jnp.maximum(vs[0], vs[1]),
                                             jnp.maximum(vs[2], vs[3])), vs[4])

                def fire(n, tau):
                    for i in range(5):
                        m = vs[i] >= tau
                        idxv = iota + (cbase + 16 * i) + off
                        plsc.store_compressed(vbuf.at[pl.ds(n, 16)], vs[i], mask=m)
                        plsc.store_compressed(ibuf.at[pl.ds(n, 16)], idxv, mask=m)
                        n = n + _popcnt(m)
                    return jax.lax.cond(n >= CAP,
                                        lambda a, b: prune(a, b, 128, 320, False),
                                        lambda a, b: (a, b), n, tau)

                return jax.lax.cond(jnp.any(mx >= tau), fire,
                                    lambda a, b: (a, b), n, tau)
            return group

        for c in range(NCH):
            if c > 0:
                pltpu.sync_copy(l_hbm.at[pl.ds(rowbase + c * CHUNK, CHUNK)], stage.at[pl.ds(0, CHUNK)])
            start = PRIME // GROUP if c == 0 else 0
            n, tau = jax.lax.fori_loop(start, CHUNK // GROUP,
                                       mk_group(c * CHUNK), (n, tau))

        def pad(j, _):
            cvb[pl.ds(j * 16, 16)] = ninf
            cib[pl.ds(j * 16, 16)] = jnp.full((16,), -1, jnp.int32)
            return 0
        jax.lax.fori_loop(0, (M + 16) // 16, pad, 0)
        prune(n, tau, 99, 128, True)
        pltpu.sync_copy(cvb.at[pl.ds(0, M)], outv_hbm.at[pl.ds(row * M, M)])
        pltpu.sync_copy(cib.at[pl.ds(0, M)], outi_hbm.at[pl.ds(row * M, M)])
        return 0

    jax.lax.fori_loop(0, RPW, row_body, 0)


def _tf_rotl(x, d):
    return (x << jnp.uint32(d)) | (x >> jnp.uint32(32 - d))


def _threefry2x32(c1):
    k0 = jnp.uint32(0)
    k1 = jnp.uint32(42)
    ks = [k0, k1, k0 ^ k1 ^ jnp.uint32(0x1BD11BDA)]
    rot = [[13, 15, 26, 6], [17, 29, 16, 24]]
    x0 = jnp.zeros_like(c1) + ks[0]
    x1 = c1 + ks[1]
    for g in range(5):
        for r in rot[g % 2]:
            x0 = x0 + x1
            x1 = _tf_rotl(x1, r)
            x1 = x1 ^ x0
        x0 = x0 + ks[(g + 1) % 3]
        x1 = x1 + ks[(g + 2) % 3] + jnp.uint32(g + 1)
    return x0 ^ x1


_P2R = 128


def _phase2_body(cv_ref, ci_ref, t_ref, p_ref, k_ref, po_ref, ids_ref):
    x0 = cv_ref[...]
    idx = ci_ref[...]
    valid = idx >= 0
    temp = t_ref[...]
    temp = jnp.where(temp < _SAMPLING_EPS, 1.0, temp)
    x = x0 / temp
    x = jnp.where(valid, x, -jnp.inf)
    kk = jnp.clip(k_ref[...], 1, V).astype(jnp.float32)

    n_gt = jnp.zeros_like(x)
    for l in range(M):
        n_gt = n_gt + (x[:, l:l + 1] > x).astype(jnp.float32)
    keep1 = (n_gt < kk) & valid
    x1 = jnp.where(keep1, x, -jnp.inf)
    mx = jnp.max(x1, axis=1, keepdims=True)
    e = jnp.exp(x1 - mx)
    s = e / jnp.sum(e, axis=1, keepdims=True)

    c = jnp.zeros_like(x)
    for l in range(M):
        xl = x1[:, l:l + 1]
        il = idx[:, l:l + 1]
        sl = s[:, l:l + 1]
        before = (xl < x1) | ((xl == x1) & (il <= idx))
        c = c + jnp.where(before, sl, 0.0)
    pmask = c <= (1.0 - p_ref[...])
    ismax = x1 == mx
    lastidx = jnp.max(jnp.where(ismax, idx, -1), axis=1, keepdims=True)
    force = ismax & (idx == lastidx)
    x2 = jnp.where(pmask & ~force, -jnp.inf, x1)
    mx2 = jnp.max(x2, axis=1, keepdims=True)
    e2 = jnp.exp(x2 - mx2)
    pfin = e2 / jnp.sum(e2, axis=1, keepdims=True)
    pfin = jnp.where(valid, pfin, 0.0)
    po_ref[...] = pfin

    row = pl.program_id(0) * _P2R + jax.lax.broadcasted_iota(jnp.int32, (_P2R, M), 0)
    flat = row * V + jnp.where(valid, idx, 0)
    bits = _threefry2x32(flat.astype(jnp.uint32))
    f = jax.lax.bitcast_convert_type((bits >> jnp.uint32(9))
                                     | jnp.uint32(0x3F800000), jnp.float32) - 1.0
    u = jnp.maximum(_TINY, f * (1.0 - _TINY) + _TINY)
    g = -jnp.log(-jnp.log(u))
    w = g + jnp.log(pfin + 1e-30)
    w = jnp.where(valid, w, -jnp.inf)
    wmax = jnp.max(w, axis=1, keepdims=True)
    sel = (w == wmax) & valid
    ids_ref[...] = jnp.min(jnp.where(sel, idx, V), axis=1, keepdims=True)


def _phase2(cv, ci, temps, ps, ks):
    grid = N // _P2R
    return pl.pallas_call(
        _phase2_body,
        grid=(grid,),
        in_specs=[pl.BlockSpec((_P2R, M), lambda i: (i, 0)),
                  pl.BlockSpec((_P2R, M), lambda i: (i, 0)),
                  pl.BlockSpec((_P2R, 1), lambda i: (i, 0)),
                  pl.BlockSpec((_P2R, 1), lambda i: (i, 0)),
                  pl.BlockSpec((_P2R, 1), lambda i: (i, 0))],
        out_specs=[pl.BlockSpec((_P2R, M), lambda i: (i, 0)),
                   pl.BlockSpec((_P2R, 1), lambda i: (i, 0))],
        out_shape=[jax.ShapeDtypeStruct((N, M), jnp.float32),
                   jax.ShapeDtypeStruct((N, 1), jnp.int32)],
    )(cv, ci, temps, ps, ks)


def _phase3(p_flat, ci):
    if "p3" not in _sc_cache:
        _sc_cache["p3"] = functools.partial(
            pl.kernel,
            mesh=_sc_mesh(),
            compiler_params=pltpu.CompilerParams(needs_layout_passes=False),
            out_type=jax.ShapeDtypeStruct((N * V,), jnp.float32),
            scratch_types=[pltpu.VMEM((100096,), jnp.float32),
                           pltpu.VMEM((M,), jnp.float32),
                           pltpu.VMEM((M,), jnp.int32)],
        )(_phase3_body)
    return _sc_cache["p3"](p_flat, ci)


def _phase3_body(p_hbm, ci_hbm, probs_hbm, zbuf, pst, ist):
    wid = jax.lax.axis_index("s") * NC + jax.lax.axis_index("c")
    zero = jnp.zeros((16,), jnp.float32)

    def zb(j, _):
        zbuf[pl.ds(j * 16, 16)] = zero
        return 0
    jax.lax.fori_loop(0, 100096 // 16, zb, 0)

    def row_body(r, _):
        row = wid * RPW + r
        pltpu.sync_copy(p_hbm.at[pl.ds(row * M, M)], pst)
        pltpu.sync_copy(ci_hbm.at[pl.ds(row * M, M)], ist)
        for j in range(M // 16):
            iv = ist[pl.ds(j * 16, 16)]
            pv = pst[pl.ds(j * 16, 16)]
            plsc.store_scatter(zbuf, [iv], pv, mask=iv >= 0)
        pltpu.sync_copy(zbuf.at[pl.ds(0, V)], probs_hbm.at[pl.ds(row * V, V)])
        for j in range(M // 16):
            iv = ist[pl.ds(j * 16, 16)]
            plsc.store_scatter(zbuf, [iv], zero, mask=iv >= 0)
        return 0

    jax.lax.fori_loop(0, RPW, row_body, 0)


def kernel(logits, temperatures, top_ps, top_ks):
    lflat = logits.astype(jnp.float32).reshape(N * V)
    cv, ci = _phase1(lflat)
    p, ids = _phase2(cv.reshape(N, M), ci.reshape(N, M),
                     temperatures.reshape(N, 1).astype(jnp.float32),
                     top_ps.reshape(N, 1).astype(jnp.float32),
                     top_ks.reshape(N, 1).astype(jnp.int32))
    probs = _phase3(p.reshape(N * M), ci)
    return ids.reshape(B, H), probs.reshape(N, V)


# speedup vs baseline: 154.7425x; 1.2056x over previous
import functools

import jax
import jax.numpy as jnp
import numpy as np
from jax.experimental import pallas as pl
from jax.experimental.pallas import tpu as pltpu
from jax.experimental.pallas import tpu_sc as plsc

B, H, V = 128, 4, 100000
N = B * H
M = 128
NC, NS = 2, 16
NW = NC * NS
RPW = N // NW
CHUNK = 20000
NCH = V // CHUNK
GROUP = 160
PRIME = 640
CAP = 512
VLEN = 768
_SAMPLING_EPS = 1e-5
_TINY = float(np.finfo(np.float32).tiny)

_sc_cache = {}


def _sc_mesh():
    return plsc.VectorSubcoreMesh(core_axis_name="c", subcore_axis_name="s")


_IOTA16 = None


_GDN = jax.lax.GatherDimensionNumbers(offset_dims=(), collapsed_slice_dims=(0,),
                                      start_index_map=(0,))


def _perm16(v, idx):
    return jax.lax.gather(v, idx[:, None], _GDN, (1,),
                          mode=jax.lax.GatherScatterMode.PROMISE_IN_BOUNDS)


def _popcnt(m):
    return plsc.all_reduce_population_count(m)[0]


def _hsplat(v):
    iota = jax.lax.iota(jnp.int32, 16)
    for d in (1, 2, 4, 8):
        v = v + _perm16(v, iota ^ d)
    return v


def _key_of(vf):
    b = jax.lax.bitcast_convert_type(vf, jnp.uint32)
    m = jnp.where(b >= jnp.uint32(0x80000000), jnp.uint32(0xFFFFFFFF),
                  jnp.uint32(0x80000000))
    return b ^ m


def _unkey(k):
    b = jnp.where(k >= jnp.uint32(0x80000000), k ^ jnp.uint32(0x80000000),
                  k ^ jnp.uint32(0xFFFFFFFF))
    return jax.lax.bitcast_convert_type(b, jnp.float32)


def _phase1(lflat):
    if "p1" not in _sc_cache:
        _sc_cache["p1"] = functools.partial(
            pl.kernel,
            mesh=_sc_mesh(),
            compiler_params=pltpu.CompilerParams(needs_layout_passes=False),
            out_type=[jax.ShapeDtypeStruct((N * M,), jnp.float32),
                      jax.ShapeDtypeStruct((N * M,), jnp.int32)],
            scratch_types=[pltpu.VMEM((20096,), jnp.float32),
                           pltpu.VMEM((VLEN,), jnp.float32),
                           pltpu.VMEM((VLEN,), jnp.int32),
                           pltpu.VMEM((VLEN,), jnp.uint32),
                           pltpu.VMEM((VLEN,), jnp.float32),
                           pltpu.VMEM((VLEN,), jnp.int32)],
        )(_phase1_body)
    return _sc_cache["p1"](lflat)


def _phase1_body(l_hbm, outv_hbm, outi_hbm, stage, vbuf, ibuf, kbuf, cvb, cib):
    wid = jax.lax.axis_index("s") * NC + jax.lax.axis_index("c")
    iota = jax.lax.iota(jnp.int32, 16)
    ninf = jnp.full((16,), -jnp.inf, jnp.float32)

    def count_ge(tkey, n):
        nv = (n + 15) // 16
        def cb(j, acc):
            k = kbuf[pl.ds(j * 16, 16)]
            live = (iota + j * 16) < n
            return acc + jnp.where(live & (k >= tkey), 1, 0)
        acc = jax.lax.fori_loop(0, nv, cb, jnp.zeros((16,), jnp.int32))
        return _hsplat(acc)

    def prune(n, tau, lo_t, hi_t, into_out):
        def kb(j, kmax):
            k = _key_of(vbuf[pl.ds(j * 16, 16)])
            kbuf[pl.ds(j * 16, 16)] = k
            live = (iota + j * 16) < n
            return jnp.maximum(kmax, jnp.where(live, k, jnp.uint32(0)))
        kmax = jax.lax.fori_loop(0, (n + 15) // 16, kb,
                                 jnp.zeros((16,), jnp.uint32))
        for d in (1, 2, 4, 8):
            g = plsc.bitcast(_perm16(plsc.bitcast(kmax, jnp.int32), iota ^ d),
                             jnp.uint32)
            kmax = jnp.maximum(kmax, g)

        def bis(_, st):
            lo, hi = st
            mid = lo + ((hi - lo) >> jnp.uint32(1))
            cm = count_ge(mid, n)
            ge = cm >= lo_t
            lo = jnp.where(ge, mid, lo)
            hi = jnp.where(ge, hi, mid)
            return lo, hi

        lo0 = _key_of(tau)
        hi0 = kmax + jnp.uint32(1)
        lo, _ = jax.lax.fori_loop(0, 22, bis, (lo0, hi0))
        tau2 = _unkey(lo)

        def comp(j, nn):
            v = vbuf[pl.ds(j * 16, 16)]
            ii = ibuf[pl.ds(j * 16, 16)]
            m = (((iota + j * 16) < n) & (v >= tau2))
            if into_out:
                plsc.store_compressed(cvb.at[pl.ds(nn, 16)], v, mask=m)
                plsc.store_compressed(cib.at[pl.ds(nn, 16)], ii, mask=m)
            else:
                plsc.store_compressed(vbuf.at[pl.ds(nn, 16)], v, mask=m)
                plsc.store_compressed(ibuf.at[pl.ds(nn, 16)], ii, mask=m)
            return nn + _popcnt(m)

        n2 = jax.lax.fori_loop(0, (n + 15) // 16, comp, 0)
        return n2, tau2

    def row_body(r, _):
        row = wid * RPW + r
        rowbase = row * V

        pltpu.sync_copy(l_hbm.at[pl.ds(rowbase, CHUNK)], stage.at[pl.ds(0, CHUNK)])

        def pb(j, _):
            vbuf[pl.ds(j * 16, 16)] = stage[pl.ds(j * 16, 16)]
            ibuf[pl.ds(j * 16, 16)] = iota + j * 16
            return 0
        jax.lax.fori_loop(0, PRIME // 16, pb, 0)
        n, tau = prune(PRIME, ninf, 128, 320, False)

        def mk_group(cbase):
            def group(g, carry):
                n, tau = carry
                off = g * GROUP
                vs = [stage[pl.ds(off + 16 * i, 16)] for i in range(10)]
                m01 = jnp.maximum(vs[0], vs[1])
                m23 = jnp.maximum(vs[2], vs[3])
                m45 = jnp.maximum(vs[4], vs[5])
                m67 = jnp.maximum(vs[6], vs[7])
                m89 = jnp.maximum(vs[8], vs[9])
                mx = jnp.maximum(jnp.maximum(jnp.maximum(m01, m23),
                                             jnp.maximum(m45, m67)), m89)

                def fire(n, tau):
                    for i in range(10):
                        m = vs[i] >= tau
                        idxv = iota + (cbase + 16 * i) + off
                        plsc.store_compressed(vbuf.at[pl.ds(n, 16)], vs[i], mask=m)
                        plsc.store_compressed(ibuf.at[pl.ds(n, 16)], idxv, mask=m)
                        n = n + _popcnt(m)
                    return jax.lax.cond(n >= CAP,
                                        lambda a, b: prune(a, b, 128, 320, False),
                                        lambda a, b: (a, b), n, tau)

                return jax.lax.cond(_popcnt(mx >= tau) > 0, fire,
                                    lambda a, b: (a, b), n, tau)
            return group

        for c in range(NCH):
            if c > 0:
                pltpu.sync_copy(l_hbm.at[pl.ds(rowbase + c * CHUNK, CHUNK)], stage.at[pl.ds(0, CHUNK)])
            start = PRIME // GROUP if c == 0 else 0
            n, tau = jax.lax.fori_loop(start, CHUNK // GROUP,
                                       mk_group(c * CHUNK), (n, tau))

        def pad(j, _):
            cvb[pl.ds(j * 16, 16)] = ninf
            cib[pl.ds(j * 16, 16)] = jnp.full((16,), -1, jnp.int32)
            return 0
        jax.lax.fori_loop(0, (M + 16) // 16, pad, 0)
        prune(n, tau, 99, 128, True)
        pltpu.sync_copy(cvb.at[pl.ds(0, M)], outv_hbm.at[pl.ds(row * M, M)])
        pltpu.sync_copy(cib.at[pl.ds(0, M)], outi_hbm.at[pl.ds(row * M, M)])
        return 0

    jax.lax.fori_loop(0, RPW, row_body, 0)


def _tf_rotl(x, d):
    return (x << jnp.uint32(d)) | (x >> jnp.uint32(32 - d))


def _threefry2x32(c1):
    k0 = jnp.uint32(0)
    k1 = jnp.uint32(42)
    ks = [k0, k1, k0 ^ k1 ^ jnp.uint32(0x1BD11BDA)]
    rot = [[13, 15, 26, 6], [17, 29, 16, 24]]
    x0 = jnp.zeros_like(c1) + ks[0]
    x1 = c1 + ks[1]
    for g in range(5):
        for r in rot[g % 2]:
            x0 = x0 + x1
            x1 = _tf_rotl(x1, r)
            x1 = x1 ^ x0
        x0 = x0 + ks[(g + 1) % 3]
        x1 = x1 + ks[(g + 2) % 3] + jnp.uint32(g + 1)
    return x0 ^ x1


_P2R = 128


def _phase2_body(cv_ref, ci_ref, t_ref, p_ref, k_ref, po_ref, ids_ref):
    x0 = cv_ref[...]
    idx = ci_ref[...]
    valid = idx >= 0
    temp = t_ref[...]
    temp = jnp.where(temp < _SAMPLING_EPS, 1.0, temp)
    x = x0 / temp
    x = jnp.where(valid, x, -jnp.inf)
    kk = jnp.clip(k_ref[...], 1, V).astype(jnp.float32)

    n_gt = jnp.zeros_like(x)
    for l in range(M):
        n_gt = n_gt + (x[:, l:l + 1] > x).astype(jnp.float32)
    keep1 = (n_gt < kk) & valid
    x1 = jnp.where(keep1, x, -jnp.inf)
    mx = jnp.max(x1, axis=1, keepdims=True)
    e = jnp.exp(x1 - mx)
    s = e / jnp.sum(e, axis=1, keepdims=True)

    c = jnp.zeros_like(x)
    for l in range(M):
        xl = x1[:, l:l + 1]
        il = idx[:, l:l + 1]
        sl = s[:, l:l + 1]
        before = (xl < x1) | ((xl == x1) & (il <= idx))
        c = c + jnp.where(before, sl, 0.0)
    pmask = c <= (1.0 - p_ref[...])
    ismax = x1 == mx
    lastidx = jnp.max(jnp.where(ismax, idx, -1), axis=1, keepdims=True)
    force = ismax & (idx == lastidx)
    x2 = jnp.where(pmask & ~force, -jnp.inf, x1)
    mx2 = jnp.max(x2, axis=1, keepdims=True)
    e2 = jnp.exp(x2 - mx2)
    pfin = e2 / jnp.sum(e2, axis=1, keepdims=True)
    pfin = jnp.where(valid, pfin, 0.0)
    po_ref[...] = pfin

    row = pl.program_id(0) * _P2R + jax.lax.broadcasted_iota(jnp.int32, (_P2R, M), 0)
    flat = row * V + jnp.where(valid, idx, 0)
    bits = _threefry2x32(flat.astype(jnp.uint32))
    f = jax.lax.bitcast_convert_type((bits >> jnp.uint32(9))
                                     | jnp.uint32(0x3F800000), jnp.float32) - 1.0
    u = jnp.maximum(_TINY, f * (1.0 - _TINY) + _TINY)
    g = -jnp.log(-jnp.log(u))
    w = g + jnp.log(pfin + 1e-30)
    w = jnp.where(valid, w, -jnp.inf)
    wmax = jnp.max(w, axis=1, keepdims=True)
    sel = (w == wmax) & valid
    ids_ref[...] = jnp.min(jnp.where(sel, idx, V), axis=1, keepdims=True)


def _phase2(cv, ci, temps, ps, ks):
    grid = N // _P2R
    return pl.pallas_call(
        _phase2_body,
        grid=(grid,),
        in_specs=[pl.BlockSpec((_P2R, M), lambda i: (i, 0)),
                  pl.BlockSpec((_P2R, M), lambda i: (i, 0)),
                  pl.BlockSpec((_P2R, 1), lambda i: (i, 0)),
                  pl.BlockSpec((_P2R, 1), lambda i: (i, 0)),
                  pl.BlockSpec((_P2R, 1), lambda i: (i, 0))],
        out_specs=[pl.BlockSpec((_P2R, M), lambda i: (i, 0)),
                   pl.BlockSpec((_P2R, 1), lambda i: (i, 0))],
        out_shape=[jax.ShapeDtypeStruct((N, M), jnp.float32),
                   jax.ShapeDtypeStruct((N, 1), jnp.int32)],
    )(cv, ci, temps, ps, ks)


def _phase3(p_flat, ci):
    if "p3" not in _sc_cache:
        _sc_cache["p3"] = functools.partial(
            pl.kernel,
            mesh=_sc_mesh(),
            compiler_params=pltpu.CompilerParams(needs_layout_passes=False),
            out_type=jax.ShapeDtypeStruct((N * V,), jnp.float32),
            scratch_types=[pltpu.VMEM((100096,), jnp.float32),
                           pltpu.VMEM((M,), jnp.float32),
                           pltpu.VMEM((M,), jnp.int32)],
        )(_phase3_body)
    return _sc_cache["p3"](p_flat, ci)


def _phase3_body(p_hbm, ci_hbm, probs_hbm, zbuf, pst, ist):
    wid = jax.lax.axis_index("s") * NC + jax.lax.axis_index("c")
    zero = jnp.zeros((16,), jnp.float32)

    def zb(j, _):
        zbuf[pl.ds(j * 16, 16)] = zero
        return 0
    jax.lax.fori_loop(0, 100096 // 16, zb, 0)

    def row_body(r, _):
        row = wid * RPW + r
        pltpu.sync_copy(p_hbm.at[pl.ds(row * M, M)], pst)
        pltpu.sync_copy(ci_hbm.at[pl.ds(row * M, M)], ist)
        for j in range(M // 16):
            iv = ist[pl.ds(j * 16, 16)]
            pv = pst[pl.ds(j * 16, 16)]
            plsc.store_scatter(zbuf, [iv], pv, mask=iv >= 0)
        pltpu.sync_copy(zbuf.at[pl.ds(0, V)], probs_hbm.at[pl.ds(row * V, V)])
        for j in range(M // 16):
            iv = ist[pl.ds(j * 16, 16)]
            plsc.store_scatter(zbuf, [iv], zero, mask=iv >= 0)
        return 0

    jax.lax.fori_loop(0, RPW, row_body, 0)


def kernel(logits, temperatures, top_ps, top_ks):
    lflat = logits.astype(jnp.float32).reshape(N * V)
    cv, ci = _phase1(lflat)
    p, ids = _phase2(cv.reshape(N, M), ci.reshape(N, M),
                     temperatures.reshape(N, 1).astype(jnp.float32),
                     top_ps.reshape(N, 1).astype(jnp.float32),
                     top_ks.reshape(N, 1).astype(jnp.int32))
    probs = _phase3(p.reshape(N * M), ci)
    return ids.reshape(B, H), probs.reshape(N, V)


# speedup vs baseline: 162.0756x; 1.0474x over previous
import functools

import jax
import jax.numpy as jnp
import numpy as np
from jax.experimental import pallas as pl
from jax.experimental.pallas import tpu as pltpu
from jax.experimental.pallas import tpu_sc as plsc

B, H, V = 128, 4, 100000
N = B * H
M = 128
NC, NS = 2, 16
NW = NC * NS
RPW = N // NW
CHUNK = 20000
NCH = V // CHUNK
GROUP = 160
PRIME = 640
CAP = 512
VLEN = 768
_SAMPLING_EPS = 1e-5
_TINY = float(np.finfo(np.float32).tiny)

_sc_cache = {}


def _sc_mesh():
    return plsc.VectorSubcoreMesh(core_axis_name="c", subcore_axis_name="s")


_IOTA16 = None


_GDN = jax.lax.GatherDimensionNumbers(offset_dims=(), collapsed_slice_dims=(0,),
                                      start_index_map=(0,))


def _perm16(v, idx):
    return jax.lax.gather(v, idx[:, None], _GDN, (1,),
                          mode=jax.lax.GatherScatterMode.PROMISE_IN_BOUNDS)


def _popcnt(m):
    return plsc.all_reduce_population_count(m)[0]


def _hsplat(v):
    iota = jax.lax.iota(jnp.int32, 16)
    for d in (1, 2, 4, 8):
        v = v + _perm16(v, iota ^ d)
    return v


def _key_of(vf):
    b = jax.lax.bitcast_convert_type(vf, jnp.uint32)
    m = jnp.where(b >= jnp.uint32(0x80000000), jnp.uint32(0xFFFFFFFF),
                  jnp.uint32(0x80000000))
    return b ^ m


def _unkey(k):
    b = jnp.where(k >= jnp.uint32(0x80000000), k ^ jnp.uint32(0x80000000),
                  k ^ jnp.uint32(0xFFFFFFFF))
    return jax.lax.bitcast_convert_type(b, jnp.float32)


def _phase1(lflat):
    if "p1" not in _sc_cache:
        _sc_cache["p1"] = functools.partial(
            pl.kernel,
            mesh=_sc_mesh(),
            compiler_params=pltpu.CompilerParams(needs_layout_passes=False),
            out_type=[jax.ShapeDtypeStruct((N * M,), jnp.float32),
                      jax.ShapeDtypeStruct((N * M,), jnp.int32)],
            scratch_types=[pltpu.VMEM((20096,), jnp.float32),
                           pltpu.VMEM((20096,), jnp.float32),
                           pltpu.VMEM((VLEN,), jnp.float32),
                           pltpu.VMEM((VLEN,), jnp.int32),
                           pltpu.VMEM((VLEN,), jnp.uint32),
                           pltpu.VMEM((VLEN,), jnp.float32),
                           pltpu.VMEM((VLEN,), jnp.int32),
                           pltpu.SemaphoreType.DMA,
                           pltpu.SemaphoreType.DMA],
        )(_phase1_body)
    return _sc_cache["p1"](lflat)


def _phase1_body(l_hbm, outv_hbm, outi_hbm, stA, stB, vbuf, ibuf, kbuf, cvb, cib, semA, semB):
    wid = jax.lax.axis_index("s") * NC + jax.lax.axis_index("c")
    iota = jax.lax.iota(jnp.int32, 16)
    ninf = jnp.full((16,), -jnp.inf, jnp.float32)

    def count_ge(tkey, n):
        nv = (n + 15) // 16
        def cb(j, acc):
            k = kbuf[pl.ds(j * 16, 16)]
            live = (iota + j * 16) < n
            return acc + jnp.where(live & (k >= tkey), 1, 0)
        acc = jax.lax.fori_loop(0, nv, cb, jnp.zeros((16,), jnp.int32))
        return _hsplat(acc)

    def prune(n, tau, lo_t, hi_t, into_out):
        def kb(j, kmax):
            k = _key_of(vbuf[pl.ds(j * 16, 16)])
            kbuf[pl.ds(j * 16, 16)] = k
            live = (iota + j * 16) < n
            return jnp.maximum(kmax, jnp.where(live, k, jnp.uint32(0)))
        kmax = jax.lax.fori_loop(0, (n + 15) // 16, kb,
                                 jnp.zeros((16,), jnp.uint32))
        for d in (1, 2, 4, 8):
            g = plsc.bitcast(_perm16(plsc.bitcast(kmax, jnp.int32), iota ^ d),
                             jnp.uint32)
            kmax = jnp.maximum(kmax, g)

        def bis(_, st):
            lo, hi = st
            mid = lo + ((hi - lo) >> jnp.uint32(1))
            cm = count_ge(mid, n)
            ge = cm >= lo_t
            lo = jnp.where(ge, mid, lo)
            hi = jnp.where(ge, hi, mid)
            return lo, hi

        lo0 = _key_of(tau)
        hi0 = kmax + jnp.uint32(1)
        lo, _ = jax.lax.fori_loop(0, 22, bis, (lo0, hi0))
        tau2 = _unkey(lo)

        def comp(j, nn):
            v = vbuf[pl.ds(j * 16, 16)]
            ii = ibuf[pl.ds(j * 16, 16)]
            m = (((iota + j * 16) < n) & (v >= tau2))
            if into_out:
                plsc.store_compressed(cvb.at[pl.ds(nn, 16)], v, mask=m)
                plsc.store_compressed(cib.at[pl.ds(nn, 16)], ii, mask=m)
            else:
                plsc.store_compressed(vbuf.at[pl.ds(nn, 16)], v, mask=m)
                plsc.store_compressed(ibuf.at[pl.ds(nn, 16)], ii, mask=m)
            return nn + _popcnt(m)

        n2 = jax.lax.fori_loop(0, (n + 15) // 16, comp, 0)
        return n2, tau2

    def row_body(r, _):
        row = wid * RPW + r
        rowbase = row * V

        def chunk_copy(c):
            st = stA if c % 2 == 0 else stB
            sem = semA if c % 2 == 0 else semB
            return pltpu.make_async_copy(
                l_hbm.at[pl.ds(rowbase + c * CHUNK, CHUNK)],
                st.at[pl.ds(0, CHUNK)], sem)

        chunk_copy(0).start()
        chunk_copy(0).wait()

        def pb(j, _):
            vbuf[pl.ds(j * 16, 16)] = stA[pl.ds(j * 16, 16)]
            ibuf[pl.ds(j * 16, 16)] = iota + j * 16
            return 0
        jax.lax.fori_loop(0, PRIME // 16, pb, 0)
        n, tau = prune(PRIME, ninf, 128, 320, False)

        def mk_group(cbase, stage):
            def group(g, carry):
                n, tau = carry
                off = g * GROUP
                vs = [stage[pl.ds(off + 16 * i, 16)] for i in range(10)]
                m01 = jnp.maximum(vs[0], vs[1])
                m23 = jnp.maximum(vs[2], vs[3])
                m45 = jnp.maximum(vs[4], vs[5])
                m67 = jnp.maximum(vs[6], vs[7])
                m89 = jnp.maximum(vs[8], vs[9])
                mx = jnp.maximum(jnp.maximum(jnp.maximum(m01, m23),
                                             jnp.maximum(m45, m67)), m89)

                def fire(n, tau):
                    for i in range(10):
                        m = vs[i] >= tau
                        idxv = iota + (cbase + 16 * i) + off
                        plsc.store_compressed(vbuf.at[pl.ds(n, 16)], vs[i], mask=m)
                        plsc.store_compressed(ibuf.at[pl.ds(n, 16)], idxv, mask=m)
                        n = n + _popcnt(m)
                    return jax.lax.cond(n >= CAP,
                                        lambda a, b: prune(a, b, 128, 320, False),
                                        lambda a, b: (a, b), n, tau)

                return jax.lax.cond(_popcnt(mx >= tau) > 0, fire,
                                    lambda a, b: (a, b), n, tau)
            return group

        for c in range(NCH):
            if c + 1 < NCH:
                chunk_copy(c + 1).start()
            if c > 0:
                chunk_copy(c).wait()
            start = PRIME // GROUP if c == 0 else 0
            n, tau = jax.lax.fori_loop(start, CHUNK // GROUP,
                                       mk_group(c * CHUNK,
                                                stA if c % 2 == 0 else stB),
                                       (n, tau))

        def pad(j, _):
            cvb[pl.ds(j * 16, 16)] = ninf
            cib[pl.ds(j * 16, 16)] = jnp.full((16,), -1, jnp.int32)
            return 0
        jax.lax.fori_loop(0, (M + 16) // 16, pad, 0)
        prune(n, tau, 99, 128, True)
        pltpu.sync_copy(cvb.at[pl.ds(0, M)], outv_hbm.at[pl.ds(row * M, M)])
        pltpu.sync_copy(cib.at[pl.ds(0, M)], outi_hbm.at[pl.ds(row * M, M)])
        return 0

    jax.lax.fori_loop(0, RPW, row_body, 0)


def _tf_rotl(x, d):
    return (x << jnp.uint32(d)) | (x >> jnp.uint32(32 - d))


def _threefry2x32(c1):
    k0 = jnp.uint32(0)
    k1 = jnp.uint32(42)
    ks = [k0, k1, k0 ^ k1 ^ jnp.uint32(0x1BD11BDA)]
    rot = [[13, 15, 26, 6], [17, 29, 16, 24]]
    x0 = jnp.zeros_like(c1) + ks[0]
    x1 = c1 + ks[1]
    for g in range(5):
        for r in rot[g % 2]:
            x0 = x0 + x1
            x1 = _tf_rotl(x1, r)
            x1 = x1 ^ x0
        x0 = x0 + ks[(g + 1) % 3]
        x1 = x1 + ks[(g + 2) % 3] + jnp.uint32(g + 1)
    return x0 ^ x1


_P2R = 128


def _phase2_body(cv_ref, ci_ref, t_ref, p_ref, k_ref, po_ref, ids_ref):
    x0 = cv_ref[...]
    idx = ci_ref[...]
    valid = idx >= 0
    temp = t_ref[...]
    temp = jnp.where(temp < _SAMPLING_EPS, 1.0, temp)
    x = x0 / temp
    x = jnp.where(valid, x, -jnp.inf)
    kk = jnp.clip(k_ref[...], 1, V).astype(jnp.float32)

    n_gt = jnp.zeros_like(x)
    for l in range(M):
        n_gt = n_gt + (x[:, l:l + 1] > x).astype(jnp.float32)
    keep1 = (n_gt < kk) & valid
    x1 = jnp.where(keep1, x, -jnp.inf)
    mx = jnp.max(x1, axis=1, keepdims=True)
    e = jnp.exp(x1 - mx)
    s = e / jnp.sum(e, axis=1, keepdims=True)

    c = jnp.zeros_like(x)
    for l in range(M):
        xl = x1[:, l:l + 1]
        il = idx[:, l:l + 1]
        sl = s[:, l:l + 1]
        before = (xl < x1) | ((xl == x1) & (il <= idx))
        c = c + jnp.where(before, sl, 0.0)
    pmask = c <= (1.0 - p_ref[...])
    ismax = x1 == mx
    lastidx = jnp.max(jnp.where(ismax, idx, -1), axis=1, keepdims=True)
    force = ismax & (idx == lastidx)
    x2 = jnp.where(pmask & ~force, -jnp.inf, x1)
    mx2 = jnp.max(x2, axis=1, keepdims=True)
    e2 = jnp.exp(x2 - mx2)
    pfin = e2 / jnp.sum(e2, axis=1, keepdims=True)
    pfin = jnp.where(valid, pfin, 0.0)
    po_ref[...] = pfin

    row = pl.program_id(0) * _P2R + jax.lax.broadcasted_iota(jnp.int32, (_P2R, M), 0)
    flat = row * V + jnp.where(valid, idx, 0)
    bits = _threefry2x32(flat.astype(jnp.uint32))
    f = jax.lax.bitcast_convert_type((bits >> jnp.uint32(9))
                                     | jnp.uint32(0x3F800000), jnp.float32) - 1.0
    u = jnp.maximum(_TINY, f * (1.0 - _TINY) + _TINY)
    g = -jnp.log(-jnp.log(u))
    w = g + jnp.log(pfin + 1e-30)
    w = jnp.where(valid, w, -jnp.inf)
    wmax = jnp.max(w, axis=1, keepdims=True)
    sel = (w == wmax) & valid
    ids_ref[...] = jnp.min(jnp.where(sel, idx, V), axis=1, keepdims=True)


def _phase2(cv, ci, temps, ps, ks):
    grid = N // _P2R
    return pl.pallas_call(
        _phase2_body,
        grid=(grid,),
        in_specs=[pl.BlockSpec((_P2R, M), lambda i: (i, 0)),
                  pl.BlockSpec((_P2R, M), lambda i: (i, 0)),
                  pl.BlockSpec((_P2R, 1), lambda i: (i, 0)),
                  pl.BlockSpec((_P2R, 1), lambda i: (i, 0)),
                  pl.BlockSpec((_P2R, 1), lambda i: (i, 0))],
        out_specs=[pl.BlockSpec((_P2R, M), lambda i: (i, 0)),
                   pl.BlockSpec((_P2R, 1), lambda i: (i, 0))],
        out_shape=[jax.ShapeDtypeStruct((N, M), jnp.float32),
                   jax.ShapeDtypeStruct((N, 1), jnp.int32)],
    )(cv, ci, temps, ps, ks)


def _phase3(p_flat, ci):
    if "p3" not in _sc_cache:
        _sc_cache["p3"] = functools.partial(
            pl.kernel,
            mesh=_sc_mesh(),
            compiler_params=pltpu.CompilerParams(needs_layout_passes=False),
            out_type=jax.ShapeDtypeStruct((N * V,), jnp.float32),
            scratch_types=[pltpu.VMEM((100096,), jnp.float32),
                           pltpu.VMEM((M,), jnp.float32),
                           pltpu.VMEM((M,), jnp.int32)],
        )(_phase3_body)
    return _sc_cache["p3"](p_flat, ci)


def _phase3_body(p_hbm, ci_hbm, probs_hbm, zbuf, pst, ist):
    wid = jax.lax.axis_index("s") * NC + jax.lax.axis_index("c")
    zero = jnp.zeros((16,), jnp.float32)

    def zb(j, _):
        zbuf[pl.ds(j * 16, 16)] = zero
        return 0
    jax.lax.fori_loop(0, 100096 // 16, zb, 0)

    def row_body(r, _):
        row = wid * RPW + r
        pltpu.sync_copy(p_hbm.at[pl.ds(row * M, M)], pst)
        pltpu.sync_copy(ci_hbm.at[pl.ds(row * M, M)], ist)
        for j in range(M // 16):
            iv = ist[pl.ds(j * 16, 16)]
            pv = pst[pl.ds(j * 16, 16)]
            plsc.store_scatter(zbuf, [iv], pv, mask=iv >= 0)
        pltpu.sync_copy(zbuf.at[pl.ds(0, V)], probs_hbm.at[pl.ds(row * V, V)])
        for j in range(M // 16):
            iv = ist[pl.ds(j * 16, 16)]
            plsc.store_scatter(zbuf, [iv], zero, mask=iv >= 0)
        return 0

    jax.lax.fori_loop(0, RPW, row_body, 0)


def kernel(logits, temperatures, top_ps, top_ks):
    lflat = logits.astype(jnp.float32).reshape(N * V)
    cv, ci = _phase1(lflat)
    p, ids = _phase2(cv.reshape(N, M), ci.reshape(N, M),
                     temperatures.reshape(N, 1).astype(jnp.float32),
                     top_ps.reshape(N, 1).astype(jnp.float32),
                     top_ks.reshape(N, 1).astype(jnp.int32))
    probs = _phase3(p.reshape(N * M), ci)
    return ids.reshape(B, H), probs.reshape(N, V)
